# Initial kernel scaffold; baseline (speedup 1.0000x reference)
import jax
import jax.numpy as jnp
from jax.experimental import pallas as pl


def kernel(inputs, token_table, pos_table):
    raise NotImplementedError("write your pallas kernel here")


# speedup vs baseline: 1.3556x; 1.3556x over previous
import functools

import jax
import jax.numpy as jnp
from jax import lax
from jax.experimental import pallas as pl
from jax.experimental.pallas import tpu as pltpu
from jax.experimental.pallas import tpu_sc as plsc

SEQ = 200
DIM = 32
NC = 2
NS = 16
NW = NC * NS

CS = 8
R = CS * SEQ
G = 64
NG = R // G


def _emb_kernel(n_rows, idx_hbm, tok_hbm, pos_hbm, out_hbm,
                pos_v, idx_v, buf, sem):
    rows_per_w = n_rows // NW
    n_chunks = rows_per_w // R

    wid = lax.axis_index("s") * NC + lax.axis_index("c")
    base0 = wid * rows_per_w

    pltpu.sync_copy(pos_hbm, pos_v)

    def chunk_body(ci, carry):
        base = base0 + ci * R
        pltpu.sync_copy(idx_hbm.at[pl.ds(base, R)], idx_v)
        copies = []
        for g in range(NG):
            copies.append(pltpu.async_copy(
                tok_hbm.at[idx_v.at[pl.ds(g * G, G)]],
                buf.at[pl.ds(g * G, G)], sem))
        for c in copies:
            c.wait()
        def seq_body(s, c2):
            def row_body(p, c3):
                row = s * SEQ + p
                plsc.addupdate(buf.at[row, pl.ds(0, 16)],
                               pos_v[p, pl.ds(0, 16)])
                plsc.addupdate(buf.at[row, pl.ds(16, 16)],
                               pos_v[p, pl.ds(16, 16)])
                return c3
            return lax.fori_loop(0, SEQ, row_body, c2)
        lax.fori_loop(0, CS, seq_body, 0)
        pltpu.sync_copy(buf, out_hbm.at[pl.ds(base, R)])
        return carry

    lax.fori_loop(0, n_chunks, chunk_body, 0)


def kernel(inputs, token_table, pos_table):
    b, s = inputs.shape
    n_rows = b * s
    assert s == SEQ and token_table.shape[1] == DIM
    assert n_rows % (NW * R) == 0

    idx = inputs.reshape(n_rows).astype(jnp.int32)

    mesh = plsc.VectorSubcoreMesh(core_axis_name="c", subcore_axis_name="s")
    k = functools.partial(
        pl.kernel,
        mesh=mesh,
        compiler_params=pltpu.CompilerParams(use_tc_tiling_on_sc=False),
        out_type=jax.ShapeDtypeStruct((n_rows, DIM), jnp.float32),
        scratch_types=[
            pltpu.VMEM((SEQ, DIM), jnp.float32),
            pltpu.VMEM((R,), jnp.int32),
            pltpu.VMEM((R, DIM), jnp.float32),
            pltpu.SemaphoreType.DMA,
        ],
    )(functools.partial(_emb_kernel, n_rows))

    out = k(idx, token_table, pos_table)
    return out.reshape(b, s, DIM)


# speedup vs baseline: 1.4491x; 1.0690x over previous
import functools

import jax
import jax.numpy as jnp
from jax import lax
from jax.experimental import pallas as pl
from jax.experimental.pallas import tpu as pltpu
from jax.experimental.pallas import tpu_sc as plsc

SEQ = 200
DIM = 32
NC = 2
NS = 16
NW = NC * NS

CS = 8
R = CS * SEQ
G = 64
NG = R // G


def _emb_kernel(n_rows, idx_hbm, tok_hbm, pos_hbm, out_hbm,
                pos_v, idx0, idx1, buf0, buf1, gs0, gs1, ws0, ws1):
    rows_per_w = n_rows // NW
    n_chunks = rows_per_w // R

    wid = lax.axis_index("s") * NC + lax.axis_index("c")
    base0 = wid * rows_per_w
    sets = ((idx0, buf0, gs0, ws0), (idx1, buf1, gs1, ws1))

    def fire(c, idxr, bufr, gsem):
        base = base0 + c * R
        pltpu.sync_copy(idx_hbm.at[pl.ds(base, R)], idxr)
        for g in range(NG):
            pltpu.async_copy(tok_hbm.at[idxr.at[pl.ds(g * G, G)]],
                             bufr.at[pl.ds(g * G, G)], gsem)

    def drain_gathers(idxr, bufr, gsem):
        for g in range(NG):
            pltpu.make_async_copy(tok_hbm.at[idxr.at[pl.ds(g * G, G)]],
                                  bufr.at[pl.ds(g * G, G)], gsem).wait()

    def wait_writeback(c, bufr, wsem):
        pltpu.make_async_copy(bufr, out_hbm.at[pl.ds(base0 + c * R, R)],
                              wsem).wait()

    pltpu.sync_copy(pos_hbm, pos_v)
    fire(0, idx0, buf0, gs0)

    def pair_body(i, carry):
        for b in (0, 1):
            idxr, bufr, gsem, wsem = sets[b]
            nidxr, nbufr, ngsem, nwsem = sets[1 - b]
            c = 2 * i + b

            @pl.when(c + 1 < n_chunks)
            def _fire_next():
                @pl.when(c >= 1)
                def _wb():
                    wait_writeback(c - 1, nbufr, nwsem)
                fire(c + 1, nidxr, nbufr, ngsem)

            drain_gathers(idxr, bufr, gsem)

            def seq_body(s, c2):
                @plsc.parallel_loop(0, SEQ, unroll=8)
                def _row(p):
                    row = s * SEQ + p
                    plsc.addupdate(bufr.at[row, pl.ds(0, 16)],
                                   pos_v[p, pl.ds(0, 16)])
                    plsc.addupdate(bufr.at[row, pl.ds(16, 16)],
                                   pos_v[p, pl.ds(16, 16)])
                return c2
            lax.fori_loop(0, CS, seq_body, 0)

            pltpu.async_copy(bufr, out_hbm.at[pl.ds(base0 + c * R, R)], wsem)
        return carry

    lax.fori_loop(0, n_chunks // 2, pair_body, 0)
    wait_writeback(n_chunks - 2, buf0, ws0)
    wait_writeback(n_chunks - 1, buf1, ws1)


def kernel(inputs, token_table, pos_table):
    b, s = inputs.shape
    n_rows = b * s
    assert s == SEQ and token_table.shape[1] == DIM
    assert n_rows % (NW * 2 * R) == 0

    idx = inputs.reshape(n_rows).astype(jnp.int32)

    mesh = plsc.VectorSubcoreMesh(core_axis_name="c", subcore_axis_name="s")
    k = functools.partial(
        pl.kernel,
        mesh=mesh,
        compiler_params=pltpu.CompilerParams(use_tc_tiling_on_sc=False),
        out_type=jax.ShapeDtypeStruct((n_rows, DIM), jnp.float32),
        scratch_types=[
            pltpu.VMEM((SEQ, DIM), jnp.float32),
            pltpu.VMEM((R,), jnp.int32),
            pltpu.VMEM((R,), jnp.int32),
            pltpu.VMEM((R, DIM), jnp.float32),
            pltpu.VMEM((R, DIM), jnp.float32),
            pltpu.SemaphoreType.DMA,
            pltpu.SemaphoreType.DMA,
            pltpu.SemaphoreType.DMA,
            pltpu.SemaphoreType.DMA,
        ],
    )(functools.partial(_emb_kernel, n_rows))

    out = k(idx, token_table, pos_table)
    return out.reshape(b, s, DIM)
